# Initial kernel scaffold; baseline (speedup 1.0000x reference)
import jax
import jax.numpy as jnp
from jax.experimental import pallas as pl


def kernel(x, edges, membership, conv_Wz, conv_Uz, conv_Wr, conv_Ur, conv_Wn, conv_Un, conv_bz, conv_br, conv_bn, r_Wz, r_Uz, r_Wr, r_Ur, r_Wn, r_Un, r_bz, r_br, r_bn, W0, b0, g0, bt0, W1, b1, g1, bt1, W2, b2):
    raise NotImplementedError("write your pallas kernel here")


# speedup vs baseline: 2.6013x; 2.6013x over previous
import functools

import jax
import jax.numpy as jnp
from jax import lax
from jax.experimental import pallas as pl
from jax.experimental.pallas import tpu as pltpu
from jax.experimental.pallas import tpu_sc as plsc

N = 10000
E = 320000
D = 128
HID = 256
G = 64
BLOCKS = 5
OUT = 2

NC = 2
NS = 16
NW = NC * NS
RPT = 320
NPAD = NW * RPT
CHUNK = 128
RC = 64
NJ = D // 16

_sc_mesh = plsc.VectorSubcoreMesh(
    core_axis_name="c", subcore_axis_name="s", num_cores=NC, num_subcores=NS)


def _wid():
  return lax.axis_index("s") * NC + lax.axis_index("c")


@functools.partial(
    pl.kernel,
    out_type=jax.ShapeDtypeStruct((NW, RPT, D), jnp.float32),
    mesh=_sc_mesh,
    scratch_types=[
        pltpu.VMEM((64,), jnp.int32),
        pltpu.VMEM((CHUNK,), jnp.int32),
        pltpu.VMEM((CHUNK + 16,), jnp.int32),
        pltpu.VMEM((CHUNK, D), jnp.float32),
        pltpu.VMEM((RPT, D), jnp.float32),
        pltpu.VMEM((RPT, D), jnp.float32),
        pltpu.SemaphoreType.DMA,
    ],
)
def _edge_agg(x_hbm, src_hbm, dst_hbm, offs_hbm, out_hbm,
              offs_v, idx_v, dst_v, rows_v, acc_s, acc_m, sem):
  w = _wid()
  base = w * RPT
  pltpu.sync_copy(offs_hbm, offs_v)
  ev = offs_v[pl.ds(w, 16)]
  e0 = ev[0]
  e1 = ev[1]

  zero16 = jnp.zeros((16,), jnp.float32)
  ninf16 = jnp.full((16,), -jnp.inf, jnp.float32)

  def init_row(i, c):
    for j in range(NJ):
      sl = pl.ds(j * 16, 16)
      acc_s[i, sl] = zero16
      acc_m[i, sl] = ninf16
    return c
  lax.fori_loop(0, RPT, init_row, 0)

  start0 = (e0 // CHUNK) * CHUNK
  nch = (e1 - start0 + CHUNK - 1) // CHUNK

  def chunk_body(k, c):
    cs = start0 + k * CHUNK
    pltpu.sync_copy(src_hbm.at[pl.ds(cs, CHUNK)], idx_v)
    pltpu.sync_copy(dst_hbm.at[pl.ds(cs, CHUNK)], dst_v.at[pl.ds(0, CHUNK)])
    pltpu.async_copy(x_hbm.at[idx_v], rows_v, sem).wait()

    def edge_body(i, c2):
      ge = cs + i
      ok = jnp.logical_and(ge >= e0, ge < e1)

      @pl.when(ok)
      def _():
        d = dst_v[pl.ds(i, 16)][0] - base
        for j in range(NJ):
          sl = pl.ds(j * 16, 16)
          r = rows_v[i, sl]
          acc_s[d, sl] = acc_s[d, sl] + r
          acc_m[d, sl] = jnp.maximum(acc_m[d, sl], r)
      return c2
    lax.fori_loop(0, CHUNK, edge_body, 0)
    return c
  lax.fori_loop(0, nch, chunk_body, 0)

  def comb_row(i, c):
    for j in range(NJ):
      sl = pl.ds(j * 16, 16)
      m = acc_m[i, sl]
      m0 = jnp.where(m == -jnp.inf, zero16, m)
      acc_s[i, sl] = acc_s[i, sl] + m0
    return c
  lax.fori_loop(0, RPT, comb_row, 0)
  pltpu.sync_copy(acc_s, out_hbm.at[w])


@functools.partial(
    pl.kernel,
    out_type=(jax.ShapeDtypeStruct((BLOCKS + 1, NW, 2, D), jnp.float32),
              jax.ShapeDtypeStruct((BLOCKS + 1, NW, 2, D), jnp.float32)),
    mesh=_sc_mesh,
    scratch_types=[
        pltpu.VMEM((NPAD,), jnp.int32),
        pltpu.VMEM((128,), jnp.int32),
        pltpu.VMEM((RC, D), jnp.float32),
        pltpu.VMEM((2, D), jnp.float32),
        pltpu.VMEM((2, D), jnp.float32),
        pltpu.SemaphoreType.DMA,
    ],
)
def _pool(mem_hbm, mboff_hbm, h0, h1, h2, h3, h4, h5, gs_hbm, gm_hbm,
          mem_v, off_v, buf, acc_s, acc_m, sem):
  t = _wid()
  pltpu.sync_copy(mem_hbm, mem_v)
  pltpu.sync_copy(mboff_hbm, off_v)
  rv = off_v[pl.ds(2 * t, 16)]
  r0 = rv[0]
  r1 = rv[2]
  gbase = 2 * t
  rs0 = (r0 // RC) * RC

  zero16 = jnp.zeros((16,), jnp.float32)
  ninf16 = jnp.full((16,), -jnp.inf, jnp.float32)
  nch = (r1 - rs0 + RC - 1) // RC

  for hh, h_hbm in enumerate((h0, h1, h2, h3, h4, h5)):
    for g in range(2):
      for j in range(NJ):
        sl = pl.ds(j * 16, 16)
        acc_s[g, sl] = zero16
        acc_m[g, sl] = ninf16

    def chunk_body(k, c, h_hbm=h_hbm):
      rs = rs0 + k * RC
      pltpu.sync_copy(h_hbm.at[pl.ds(rs, RC)], buf)

      def row_body(i, c2):
        gr = rs + i
        ok = jnp.logical_and(gr >= r0, gr < r1)

        @pl.when(ok)
        def _():
          g = mem_v[pl.ds(gr, 16)][0] - gbase
          for j in range(NJ):
            sl = pl.ds(j * 16, 16)
            r = buf[i, sl]
            acc_s[g, sl] = acc_s[g, sl] + r
            acc_m[g, sl] = jnp.maximum(acc_m[g, sl], r)
        return c2
      lax.fori_loop(0, RC, row_body, 0)
      return c
    lax.fori_loop(0, nch, chunk_body, 0)

    for g in range(2):
      for j in range(NJ):
        sl = pl.ds(j * 16, 16)
        m = acc_m[g, sl]
        acc_m[g, sl] = jnp.where(m == -jnp.inf, zero16, m)
    pltpu.sync_copy(acc_s, gs_hbm.at[hh].at[t])
    pltpu.sync_copy(acc_m, gm_hbm.at[hh].at[t])


GRID_ROWS = 2560


def _gru_body(agg_ref, inp_ref, w_ref, u_ref, b_ref, h_ref, out_ref):
  a = agg_ref[...]
  x = inp_ref[...]
  A = jnp.dot(a, w_ref[...], preferred_element_type=jnp.float32)
  B = jnp.dot(x, u_ref[...], preferred_element_type=jnp.float32)
  z = jax.nn.sigmoid(A[:, 0:D] + B[:, 0:D] + b_ref[0:1, :])
  r = jax.nn.sigmoid(A[:, D:2 * D] + B[:, D:2 * D] + b_ref[1:2, :])
  n = jnp.tanh(A[:, 2 * D:3 * D] + r * B[:, 2 * D:3 * D] + b_ref[2:3, :])
  h = (1.0 - z) * n + z * x
  h_ref[...] = h
  out_ref[...] = x + h


def _gru_call(agg, inp, wcat, ucat, bcat):
  blk = pl.BlockSpec((GRID_ROWS, D), lambda i: (i, 0))
  wspec = pl.BlockSpec((D, 3 * D), lambda i: (0, 0))
  bspec = pl.BlockSpec((3, D), lambda i: (0, 0))
  return pl.pallas_call(
      _gru_body,
      grid=(NPAD // GRID_ROWS,),
      in_specs=[blk, blk, wspec, wspec, bspec],
      out_specs=[blk, blk],
      out_shape=[jax.ShapeDtypeStruct((NPAD, D), jnp.float32),
                 jax.ShapeDtypeStruct((NPAD, D), jnp.float32)],
  )(agg, inp, wcat, ucat, bcat)


def _readout_body(gs_ref, gm_ref, rw_ref, ru_ref, rb_ref,
                  w0_ref, p0_ref, w1_ref, p1_ref, w2_ref, b2_ref,
                  logits_ref, graphs_ref):
  ps = gs_ref[...]
  pm = gm_ref[...]
  A = jnp.dot(ps, rw_ref[...], preferred_element_type=jnp.float32)
  B = jnp.dot(pm, ru_ref[...], preferred_element_type=jnp.float32)
  z = jax.nn.sigmoid(A[:, 0:D] + B[:, 0:D] + rb_ref[0:1, :])
  r = jax.nn.sigmoid(A[:, D:2 * D] + B[:, D:2 * D] + rb_ref[1:2, :])
  n = jnp.tanh(A[:, 2 * D:3 * D] + r * B[:, 2 * D:3 * D] + rb_ref[2:3, :])
  pooled = (1.0 - z) * n + z * pm

  for i in range(BLOCKS + 1):
    graphs_ref[:, i * D:(i + 1) * D] = pooled[i * G:(i + 1) * G, :]
  graphs = graphs_ref[...]

  def bn_gelu(v, p_ref):
    g = p_ref[0:1, :]
    bt = p_ref[1:2, :]
    b = p_ref[2:3, :]
    v = v + b
    mu = jnp.mean(v, axis=0, keepdims=True)
    var = jnp.mean((v - mu) ** 2, axis=0, keepdims=True)
    vn = g * (v - mu) / jnp.sqrt(var + 1e-5) + bt
    return jax.nn.gelu(vn)

  h0 = bn_gelu(jnp.dot(graphs, w0_ref[...], preferred_element_type=jnp.float32),
               p0_ref)
  h1 = bn_gelu(jnp.dot(h0, w1_ref[...], preferred_element_type=jnp.float32),
               p1_ref)
  logits_ref[...] = (
      jnp.dot(h1, w2_ref[...], preferred_element_type=jnp.float32)
      + b2_ref[0:1, :])


def _readout_call(gs, gm, rwcat, rucat, rbcat, w0, p0, w1, p1, w2, b2):
  return pl.pallas_call(
      _readout_body,
      out_shape=[jax.ShapeDtypeStruct((G, OUT), jnp.float32),
                 jax.ShapeDtypeStruct((G, D * (BLOCKS + 1)), jnp.float32)],
  )(gs, gm, rwcat, rucat, rbcat, w0, p0, w1, p1, w2, b2)


def kernel(x, edges, membership, conv_Wz, conv_Uz, conv_Wr, conv_Ur, conv_Wn,
           conv_Un, conv_bz, conv_br, conv_bn, r_Wz, r_Uz, r_Wr, r_Ur, r_Wn,
           r_Un, r_bz, r_br, r_bn, W0, b0, g0, bt0, W1, b1, g1, bt1, W2, b2):
  src = edges[0]
  dst = edges[1]
  dst_s, src_s = lax.sort((dst, src), num_keys=1)

  bounds = (jnp.arange(33, dtype=jnp.int32) * RPT).astype(jnp.int32)
  offs = jnp.searchsorted(dst_s, bounds).astype(jnp.int32)
  offs = jnp.concatenate([offs, jnp.zeros((31,), jnp.int32)])

  mb = jnp.searchsorted(membership,
                        jnp.arange(65, dtype=jnp.int32)).astype(jnp.int32)
  mb = jnp.concatenate([mb, jnp.zeros((63,), jnp.int32)])

  x_pad = jnp.pad(x, ((0, NPAD - N), (0, 0)))

  wcat = jnp.concatenate([conv_Wz, conv_Wr, conv_Wn], axis=2)
  ucat = jnp.concatenate([conv_Uz, conv_Ur, conv_Un], axis=2)
  bcat = jnp.stack([conv_bz, conv_br, conv_bn], axis=1)

  hiddens = [x_pad]
  block_input = x_pad
  mem_pad = jnp.pad(membership, (0, NPAD - N))

  for i in range(BLOCKS):
    agg = _edge_agg(block_input, src_s, dst_s, offs).reshape(NPAD, D)
    h, block_input = _gru_call(agg, block_input, wcat[i], ucat[i], bcat[i])
    hiddens.append(h)

  gs, gm = _pool(mem_pad, mb, *hiddens)
  gs = gs.reshape((BLOCKS + 1) * G, D)
  gm = gm.reshape((BLOCKS + 1) * G, D)

  rwcat = jnp.concatenate([r_Wz, r_Wr, r_Wn], axis=1)
  rucat = jnp.concatenate([r_Uz, r_Ur, r_Un], axis=1)
  rbcat = jnp.stack([r_bz, r_br, r_bn], axis=0)
  p0 = jnp.stack([g0, bt0, b0], axis=0)
  p1 = jnp.stack([g1, bt1, b1], axis=0)
  b2r = b2.reshape(1, OUT)

  logits, graphs = _readout_call(gs, gm, rwcat, rucat, rbcat,
                                 W0, p0, W1, p1, W2, b2r)
  return (logits, graphs)
